# Initial kernel scaffold; baseline (speedup 1.0000x reference)
import jax
import jax.numpy as jnp
from jax.experimental import pallas as pl


def kernel(train_inputs, train_types, node_neigh, node_embeddings, node_type_embeddings, trans_weights, trans_weights_s1, trans_weights_s2):
    raise NotImplementedError("write your pallas kernel here")


# speedup vs baseline: 51.7870x; 51.7870x over previous
import functools

import jax
import jax.numpy as jnp
from jax import lax
from jax.experimental import pallas as pl
from jax.experimental.pallas import tpu as pltpu
from jax.experimental.pallas import tpu_sc as plsc

NUM_NODES = 100000
EMB = 128
EMB_U = 32
T = 4
B = 8192
NEIGH = 10

NW = 32
SEGS = B * T
SEG_W = SEGS // NW
GSEG = 8
GIDX = GSEG * NEIGH
NG_W = SEG_W // GSEG

BLK = 1024


def _sc_segment_sum(idx2d, table2d):
    mesh = plsc.VectorSubcoreMesh(core_axis_name="c", subcore_axis_name="s")

    @functools.partial(
        pl.kernel,
        mesh=mesh,
        out_type=jax.ShapeDtypeStruct((SEGS, EMB_U), jnp.float32),
        scratch_types=[
            pltpu.VMEM((NG_W, GIDX), jnp.int32),
            pltpu.VMEM((GIDX, EMB_U), jnp.float32),
            pltpu.VMEM((SEG_W, EMB_U), jnp.float32),
            pltpu.SemaphoreType.DMA,
        ],
        compiler_params=pltpu.CompilerParams(use_tc_tiling_on_sc=False),
    )
    def k(idx_hbm, table_hbm, out_hbm, idx_v, rows_v, out_v, sem):
        wid = lax.axis_index("s") * 2 + lax.axis_index("c")
        pltpu.sync_copy(idx_hbm.at[pl.ds(wid * NG_W, NG_W)], idx_v)

        def body(g, carry):
            pltpu.async_copy(table_hbm.at[idx_v.at[g]], rows_v, sem).wait()
            for s in range(GSEG):
                base = s * NEIGH
                a0 = rows_v[base, 0:16]
                a1 = rows_v[base, 16:32]
                for n in range(1, NEIGH):
                    a0 = a0 + rows_v[base + n, 0:16]
                    a1 = a1 + rows_v[base + n, 16:32]
                seg = g * GSEG + s
                out_v[seg, 0:16] = a0
                out_v[seg, 16:32] = a1
            return carry

        lax.fori_loop(0, NG_W, body, 0)
        pltpu.sync_copy(out_v, out_hbm.at[pl.ds(wid * SEG_W, SEG_W)])

    return k(idx2d, table2d)


def _tc_dense_body(x_ref, t_ref, w1_ref, w2_ref, w_ref, o_ref):
    tb = t_ref[...]
    xt = [x_ref[:, i * EMB_U:(i + 1) * EMB_U] for i in range(T)]
    oh = [(tb == s).astype(jnp.float32) for s in range(T)]

    logits = [jnp.zeros((BLK, 1), jnp.float32) for _ in range(T)]
    for s in range(T):
        w1s = w1_ref[s]
        w2s = w2_ref[s][:, 0][None, :]
        for i in range(T):
            h = jnp.tanh(jnp.dot(xt[i], w1s, preferred_element_type=jnp.float32))
            l = jnp.sum(h * w2s, axis=1, keepdims=True)
            logits[i] = logits[i] + oh[s] * l

    m = logits[0]
    for i in range(1, T):
        m = jnp.maximum(m, logits[i])
    e = [jnp.exp(l - m) for l in logits]
    z = e[0]
    for i in range(1, T):
        z = z + e[i]
    inv_z = 1.0 / z

    agg = jnp.zeros((BLK, EMB_U), jnp.float32)
    for i in range(T):
        agg = agg + (e[i] * inv_z) * xt[i]

    out = jnp.zeros((BLK, EMB), jnp.float32)
    for s in range(T):
        out = out + oh[s] * jnp.dot(agg, w_ref[s], preferred_element_type=jnp.float32)

    nrm = jnp.maximum(jnp.sqrt(jnp.sum(out * out, axis=1, keepdims=True)), 1e-12)
    o_ref[...] = out / nrm


def _tc_dense(nte, types2d, w1, w2, w):
    grid = (B // BLK,)
    return pl.pallas_call(
        _tc_dense_body,
        grid=grid,
        in_specs=[
            pl.BlockSpec((BLK, T * EMB_U), lambda i: (i, 0)),
            pl.BlockSpec((BLK, 1), lambda i: (i, 0)),
            pl.BlockSpec((T, EMB_U, EMB_U), lambda i: (0, 0, 0)),
            pl.BlockSpec((T, EMB_U, 1), lambda i: (0, 0, 0)),
            pl.BlockSpec((T, EMB_U, EMB), lambda i: (0, 0, 0)),
        ],
        out_specs=pl.BlockSpec((BLK, EMB), lambda i: (i, 0)),
        out_shape=jax.ShapeDtypeStruct((B, EMB), jnp.float32),
    )(nte, types2d, w1, w2, w)


def kernel(train_inputs, train_types, node_neigh, node_embeddings,
           node_type_embeddings, trans_weights, trans_weights_s1, trans_weights_s2):
    del train_inputs, node_embeddings
    tvec = jnp.arange(T, dtype=node_neigh.dtype)
    idx2d = (node_neigh * T + tvec[None, :, None]).reshape(SEGS * NEIGH // GIDX, GIDX)
    idx2d = idx2d.astype(jnp.int32)
    table2d = node_type_embeddings.reshape(NUM_NODES * T, EMB_U)

    nte = _sc_segment_sum(idx2d, table2d)
    nte = nte.reshape(B, T * EMB_U)

    out = _tc_dense(nte, train_types.reshape(B, 1).astype(jnp.int32),
                    trans_weights_s1, trans_weights_s2, trans_weights)
    return out
